# baseline (speedup 1.0000x reference)
import functools

import jax
import jax.numpy as jnp
from jax import lax
from jax.experimental import pallas as pl
from jax.experimental.pallas import tpu as pltpu
from jax.experimental.pallas import tpu_sc as plsc

F = 64
K = 16
BN = 2048
NC = 2
NS = 16
NW = NC * NS
CA = 64
IDX_ROWS = (CA * K) // 128


def _full_spec(shape):
    return pl.BlockSpec(shape, lambda i: (0,) * len(shape))


def _row_spec(cols):
    return pl.BlockSpec((BN, cols), lambda i: (i, 0))


def _embed_pre_body(af, Wa, ba, W1a, b1, W1b, x_o, sp_o, y_o):
    x = jnp.dot(af[...], Wa[...], preferred_element_type=jnp.float32) + ba[...]
    x_o[...] = x
    sp_o[...] = jnp.dot(x, W1a[...], preferred_element_type=jnp.float32) + b1[...]
    y = jnp.dot(x, W1b[...], preferred_element_type=jnp.float32)
    y_o[...] = jnp.concatenate([y, jnp.zeros_like(y)], axis=-1)


def _update(x, s, Wm2, bm2, Wu1a, Wu1b, bu1, Wu2, bu2):
    agg = jnp.dot(s, Wm2, preferred_element_type=jnp.float32) + 16.0 * bm2
    h = jnp.maximum(
        jnp.dot(x, Wu1a, preferred_element_type=jnp.float32)
        + jnp.dot(agg, Wu1b, preferred_element_type=jnp.float32) + bu1, 0.0)
    upd = jnp.dot(h, Wu2, preferred_element_type=jnp.float32) + bu2
    return jnp.maximum(x + upd, 0.0)


def _mid_body(x, s, Wm2, bm2, Wu1a, Wu1b, bu1, Wu2, bu2, W1a, b1, W1b,
              xn_o, sp_o, y_o):
    xn = _update(x[...], s[...], Wm2[...], bm2[...], Wu1a[...], Wu1b[...],
                 bu1[...], Wu2[...], bu2[...])
    xn_o[...] = xn
    sp_o[...] = jnp.dot(xn, W1a[...], preferred_element_type=jnp.float32) + b1[...]
    y = jnp.dot(xn, W1b[...], preferred_element_type=jnp.float32)
    y_o[...] = jnp.concatenate([y, jnp.zeros_like(y)], axis=-1)


def _final_body(x, s, Wm2, bm2, Wu1a, Wu1b, bu1, Wu2, bu2,
                We1, be1, We2, be2, Wp, bp, props_o, feat_o, xn_o):
    xn = _update(x[...], s[...], Wm2[...], bm2[...], Wu1a[...], Wu1b[...],
                 bu1[...], Wu2[...], bu2[...])
    f1 = jnp.maximum(
        jnp.dot(xn, We1[...], preferred_element_type=jnp.float32) + be1[...], 0.0)
    feat = jnp.dot(f1, We2[...], preferred_element_type=jnp.float32) + be2[...]
    props_o[...] = jnp.dot(feat, Wp[...], preferred_element_type=jnp.float32) + bp[...]
    feat_o[...] = feat
    xn_o[...] = xn


def _tc_embed_pre(n_pad, af, Wa, ba, W1a, b1, W1b):
    nb = n_pad // BN
    out = jax.ShapeDtypeStruct((n_pad, F), jnp.float32)
    out_y = jax.ShapeDtypeStruct((n_pad, 128), jnp.float32)
    return pl.pallas_call(
        _embed_pre_body,
        grid=(nb,),
        in_specs=[_row_spec(128), _full_spec((128, F)), _full_spec((1, F)),
                  _full_spec((F, F)), _full_spec((1, F)), _full_spec((F, F))],
        out_specs=[_row_spec(F), _row_spec(F), _row_spec(128)],
        out_shape=[out, out, out_y],
    )(af, Wa, ba, W1a, b1, W1b)


def _tc_mid(n_pad, x, s, Wm2, bm2, Wu1a, Wu1b, bu1, Wu2, bu2, W1a, b1, W1b):
    nb = n_pad // BN
    out = jax.ShapeDtypeStruct((n_pad, F), jnp.float32)
    out_y = jax.ShapeDtypeStruct((n_pad, 128), jnp.float32)
    wspec = _full_spec((F, F))
    bspec = _full_spec((1, F))
    return pl.pallas_call(
        _mid_body,
        grid=(nb,),
        in_specs=[_row_spec(F), _row_spec(F),
                  wspec, bspec, wspec, wspec, bspec, wspec, bspec,
                  wspec, bspec, wspec],
        out_specs=[_row_spec(F), _row_spec(F), _row_spec(128)],
        out_shape=[out, out, out_y],
    )(x, s, Wm2, bm2, Wu1a, Wu1b, bu1, Wu2, bu2, W1a, b1, W1b)


def _tc_final(n_pad, x, s, Wm2, bm2, Wu1a, Wu1b, bu1, Wu2, bu2,
              We1, be1, We2, be2, Wp, bp):
    nb = n_pad // BN
    wspec = _full_spec((F, F))
    bspec = _full_spec((1, F))
    return pl.pallas_call(
        _final_body,
        grid=(nb,),
        in_specs=[_row_spec(F), _row_spec(F),
                  wspec, bspec, wspec, wspec, bspec, wspec, bspec,
                  _full_spec((F, 128)), _full_spec((1, 128)),
                  _full_spec((128, F)), bspec,
                  _full_spec((F, 4)), _full_spec((1, 4))],
        out_specs=[_row_spec(4), _row_spec(F), _row_spec(F)],
        out_shape=[jax.ShapeDtypeStruct((n_pad, 4), jnp.float32),
                   jax.ShapeDtypeStruct((n_pad, F), jnp.float32),
                   jax.ShapeDtypeStruct((n_pad, F), jnp.float32)],
    )(x, s, Wm2, bm2, Wu1a, Wu1b, bu1, Wu2, bu2, We1, be1, We2, be2, Wp, bp)


def _sc_body(G0, G1, y_hbm, sp_hbm, idx_hbm, s_hbm,
             i0, i1, r0, r1, sp0, sp1, ac0, ac1,
             isem, g0, g1, sps, o0, o1):
    c = lax.axis_index("c")
    s = lax.axis_index("s")
    is0 = c == 0
    groups = jnp.where(is0, G0, G1)
    base = jnp.where(is0, s * (G0 * CA), NS * G0 * CA + s * (G1 * CA))

    ibufs, rbufs, gsems = [i0, i1], [r0, r1], [g0, g1]
    spbufs, acbufs, osems = [sp0, sp1], [ac0, ac1], [o0, o1]
    WA = 256

    def issue_idx(g, p):
        rb = pl.multiple_of((base + g * CA) // 8, 8)
        pltpu.async_copy(idx_hbm.at[pl.ds(rb, CA * K // 128), :], ibufs[p],
                         isem)

    def wait_idx(p):
        pltpu.make_async_copy(idx_hbm.at[pl.ds(0, CA * K // 128), :],
                              ibufs[p], isem).wait()

    def issue_wave(ip, pair, p):
        pltpu.async_copy(y_hbm.at[ibufs[ip].at[2 * pair]],
                         rbufs[p].at[pl.ds(0, 128), :], gsems[p])
        pltpu.async_copy(y_hbm.at[ibufs[ip].at[2 * pair + 1]],
                         rbufs[p].at[pl.ds(128, 128), :], gsems[p])

    def wait_wave(p):
        pltpu.make_async_copy(y_hbm.at[pl.ds(0, WA), :], rbufs[p],
                              gsems[p]).wait()

    def issue_sp(g, p):
        ab = pl.multiple_of(base + g * CA, CA)
        pltpu.async_copy(sp_hbm.at[pl.ds(ab, CA), :], spbufs[p], sps)

    def wait_sp(p):
        pltpu.make_async_copy(sp_hbm.at[pl.ds(0, CA), :], spbufs[p], sps).wait()

    def issue_out(g, p):
        ab = pl.multiple_of(base + g * CA, CA)
        pltpu.async_copy(acbufs[p], s_hbm.at[pl.ds(ab, CA), :], osems[p])

    def wait_out(p):
        pltpu.make_async_copy(acbufs[p], s_hbm.at[pl.ds(0, CA), :],
                              osems[p]).wait()

    def compute_wave(w, wp, gp):
        rb, spb, acb = rbufs[wp], spbufs[gp], acbufs[gp]

        def atom_body(ai, _):
            a = w * 16 + ai
            sp = [spb[a, pl.ds(c * 16, 16)] for c in range(4)]
            acc = [jnp.zeros((16,), jnp.float32) for _ in range(4)]
            rbase = ai * K
            for k in range(K):
                for c in range(4):
                    row = rb[rbase + k, pl.ds(c * 16, 16)]
                    acc[c] = acc[c] + jnp.maximum(sp[c] + row, 0.0)
            for c in range(4):
                acb[a, pl.ds(c * 16, 16)] = acc[c]
            return 0

        lax.fori_loop(0, 16, atom_body, 0)

    def do_group(g, gp, last):
        wait_sp(gp)
        if not last:
            issue_idx(g + 1, 1 - gp)
            issue_sp(g + 1, 1 - gp)
        for w in range(4):
            wp = w % 2
            if w < 3:
                issue_wave(gp, w + 1, 1 - wp)
            elif not last:
                wait_idx(1 - gp)
                issue_wave(1 - gp, 0, 1 - wp)
            wait_wave(wp)
            compute_wave(w, wp, gp)
        issue_out(g, gp)

    issue_idx(0, 0)
    wait_idx(0)
    issue_sp(0, 0)
    issue_wave(0, 0, 0)

    def pair_body(t, _):
        @pl.when(t > 0)
        def _():
            wait_out(0)

        do_group(2 * t, 0, False)

        @pl.when(t > 0)
        def _():
            wait_out(1)

        do_group(2 * t + 1, 1, False)
        return 0

    lax.fori_loop(0, (groups - 1) // 2, pair_body, 0)

    @pl.when(groups > 1)
    def _():
        wait_out(0)

    do_group(groups - 1, 0, True)

    @pl.when(groups > 1)
    def _():
        wait_out(1)

    wait_out(0)


G_C0 = 7
G_C1 = 43


def _sc_gather_relu_sum(n_pad, y, sp, idx2d):
    total_groups = n_pad // CA
    assert NS * (G_C0 + G_C1) == total_groups
    assert G_C0 % 2 == 1 and G_C1 % 2 == 1
    mesh = plsc.VectorSubcoreMesh(core_axis_name="c", subcore_axis_name="s")
    return pl.kernel(
        functools.partial(_sc_body, G_C0, G_C1),
        out_type=jax.ShapeDtypeStruct((n_pad, F), jnp.float32),
        mesh=mesh,
        scratch_types=[
            pltpu.VMEM((CA * K // 128, 128), jnp.int32),
            pltpu.VMEM((CA * K // 128, 128), jnp.int32),
            pltpu.VMEM((256, 128), jnp.float32),
            pltpu.VMEM((256, 128), jnp.float32),
            pltpu.VMEM((CA, F), jnp.float32),
            pltpu.VMEM((CA, F), jnp.float32),
            pltpu.VMEM((CA, F), jnp.float32),
            pltpu.VMEM((CA, F), jnp.float32),
            pltpu.SemaphoreType.DMA,
            pltpu.SemaphoreType.DMA,
            pltpu.SemaphoreType.DMA,
            pltpu.SemaphoreType.DMA,
            pltpu.SemaphoreType.DMA,
            pltpu.SemaphoreType.DMA,
        ],
    )(y, sp, idx2d)


def kernel(atom_fea, nbr_fea, nbr_fea_idx, params):
    n = atom_fea.shape[0]
    n_pad = -(-n // (NW * CA)) * (NW * CA)
    assert n_pad % BN == 0

    af = jnp.pad(atom_fea, ((0, n_pad - n), (0, 128 - atom_fea.shape[1])))
    Wa = jnp.pad(params['W_atom'], ((0, 128 - params['W_atom'].shape[0]), (0, 0)))
    ba = params['b_atom'][None, :]
    idx2d = jnp.pad(nbr_fea_idx, ((0, n_pad - n), (0, 0))).reshape(-1, 128)

    def pre_w(l):
        c = params['conv%d' % l]
        return c['Wm1'][:F], c['bm1'][None, :], c['Wm1'][F:]

    def post_w(l):
        c = params['conv%d' % l]
        return (c['Wm2'], c['bm2'][None, :], c['Wu1'][:F], c['Wu1'][F:],
                c['bu1'][None, :], c['Wu2'], c['bu2'][None, :])

    x, sp, y = _tc_embed_pre(n_pad, af, Wa, ba, *pre_w(0))
    for l in range(2):
        s = _sc_gather_relu_sum(n_pad, y, sp, idx2d)
        x, sp, y = _tc_mid(n_pad, x, s, *post_w(l), *pre_w(l + 1))
    s = _sc_gather_relu_sum(n_pad, y, sp, idx2d)
    props, feat, xn = _tc_final(
        n_pad, x, s, *post_w(2),
        params['We1'], params['be1'][None, :],
        params['We2'], params['be2'][None, :],
        params['Wp'], params['bp'][None, :])

    return jnp.concatenate([props[:n], feat[:n], xn[:n]], axis=-1)


# speedup vs baseline: 1.9809x; 1.9809x over previous
import functools

import jax
import jax.numpy as jnp
import numpy as np
from jax import lax
from jax.experimental import pallas as pl
from jax.experimental.pallas import tpu as pltpu
from jax.experimental.pallas import tpu_sc as plsc

F = 64
K = 16
BN = 2048
NC = 2
NS = 16
NW = NC * NS
CA = 64
IDX_ROWS = (CA * K) // 128


def _full_spec(shape):
    return pl.BlockSpec(shape, lambda i: (0,) * len(shape))


def _row_spec(cols):
    return pl.BlockSpec((BN, cols), lambda i: (i, 0))


def _embed_pre_body(af, Wa, ba, W1a, b1, W1b, x_o, sp_o, y_o):
    x = jnp.dot(af[...], Wa[...], preferred_element_type=jnp.float32) + ba[...]
    x_o[...] = x
    sp_o[...] = jnp.dot(x, W1a[...], preferred_element_type=jnp.float32) + b1[...]
    y_o[...] = jnp.dot(x, W1b[...], preferred_element_type=jnp.float32)


def _update(x, s, Wm2, bm2, Wu1a, Wu1b, bu1, Wu2, bu2):
    agg = jnp.dot(s, Wm2, preferred_element_type=jnp.float32) + 16.0 * bm2
    h = jnp.maximum(
        jnp.dot(x, Wu1a, preferred_element_type=jnp.float32)
        + jnp.dot(agg, Wu1b, preferred_element_type=jnp.float32) + bu1, 0.0)
    upd = jnp.dot(h, Wu2, preferred_element_type=jnp.float32) + bu2
    return jnp.maximum(x + upd, 0.0)


def _mid_body(x, s, Wm2, bm2, Wu1a, Wu1b, bu1, Wu2, bu2, W1a, b1, W1b,
              xn_o, sp_o, y_o):
    xn = _update(x[...], s[...], Wm2[...], bm2[...], Wu1a[...], Wu1b[...],
                 bu1[...], Wu2[...], bu2[...])
    xn_o[...] = xn
    sp_o[...] = jnp.dot(xn, W1a[...], preferred_element_type=jnp.float32) + b1[...]
    y_o[...] = jnp.dot(xn, W1b[...], preferred_element_type=jnp.float32)


def _final_body(x, s, Wm2, bm2, Wu1a, Wu1b, bu1, Wu2, bu2,
                We1, be1, We2, be2, Wp, bp, props_o, feat_o, xn_o):
    xn = _update(x[...], s[...], Wm2[...], bm2[...], Wu1a[...], Wu1b[...],
                 bu1[...], Wu2[...], bu2[...])
    f1 = jnp.maximum(
        jnp.dot(xn, We1[...], preferred_element_type=jnp.float32) + be1[...], 0.0)
    feat = jnp.dot(f1, We2[...], preferred_element_type=jnp.float32) + be2[...]
    props_o[...] = jnp.dot(feat, Wp[...], preferred_element_type=jnp.float32) + bp[...]
    feat_o[...] = feat
    xn_o[...] = xn


def _tc_embed_pre(n_pad, af, Wa, ba, W1a, b1, W1b):
    nb = n_pad // BN
    out = jax.ShapeDtypeStruct((n_pad, F), jnp.float32)
    out_y = jax.ShapeDtypeStruct((n_pad, F), jnp.float32)
    return pl.pallas_call(
        _embed_pre_body,
        grid=(nb,),
        in_specs=[_row_spec(128), _full_spec((128, F)), _full_spec((1, F)),
                  _full_spec((F, F)), _full_spec((1, F)), _full_spec((F, F))],
        out_specs=[_row_spec(F), _row_spec(F), _row_spec(F)],
        out_shape=[out, out, out_y],
    )(af, Wa, ba, W1a, b1, W1b)


def _tc_mid(n_pad, x, s, Wm2, bm2, Wu1a, Wu1b, bu1, Wu2, bu2, W1a, b1, W1b):
    nb = n_pad // BN
    out = jax.ShapeDtypeStruct((n_pad, F), jnp.float32)
    out_y = jax.ShapeDtypeStruct((n_pad, F), jnp.float32)
    wspec = _full_spec((F, F))
    bspec = _full_spec((1, F))
    return pl.pallas_call(
        _mid_body,
        grid=(nb,),
        in_specs=[_row_spec(F), _row_spec(F),
                  wspec, bspec, wspec, wspec, bspec, wspec, bspec,
                  wspec, bspec, wspec],
        out_specs=[_row_spec(F), _row_spec(F), _row_spec(F)],
        out_shape=[out, out, out_y],
    )(x, s, Wm2, bm2, Wu1a, Wu1b, bu1, Wu2, bu2, W1a, b1, W1b)


def _tc_final(n_pad, x, s, Wm2, bm2, Wu1a, Wu1b, bu1, Wu2, bu2,
              We1, be1, We2, be2, Wp, bp):
    nb = n_pad // BN
    wspec = _full_spec((F, F))
    bspec = _full_spec((1, F))
    return pl.pallas_call(
        _final_body,
        grid=(nb,),
        in_specs=[_row_spec(F), _row_spec(F),
                  wspec, bspec, wspec, wspec, bspec, wspec, bspec,
                  _full_spec((F, 128)), _full_spec((1, 128)),
                  _full_spec((128, F)), bspec,
                  _full_spec((F, 4)), _full_spec((1, 4))],
        out_specs=[_row_spec(4), _row_spec(F), _row_spec(F)],
        out_shape=[jax.ShapeDtypeStruct((n_pad, 4), jnp.float32),
                   jax.ShapeDtypeStruct((n_pad, F), jnp.float32),
                   jax.ShapeDtypeStruct((n_pad, F), jnp.float32)],
    )(x, s, Wm2, bm2, Wu1a, Wu1b, bu1, Wu2, bu2, We1, be1, We2, be2, Wp, bp)


def _sc_body(G0, G1, y_hbm, sp_hbm, idx_hbm, s_hbm,
             i0, i1, r0, r1, sp0, sp1, ac0, ac1,
             isem, g0, g1, sps, o0, o1):
    c = lax.axis_index("c")
    s = lax.axis_index("s")
    is0 = c == 0
    groups = jnp.where(is0, G0, G1)
    base = jnp.where(is0, s * (G0 * CA), NS * G0 * CA + s * (G1 * CA))

    ibufs, rbufs, gsems = [i0, i1], [r0, r1], [g0, g1]
    spbufs, acbufs, osems = [sp0, sp1], [ac0, ac1], [o0, o1]
    WA = 256

    def issue_idx(g, p):
        rb = pl.multiple_of((base + g * CA) // 8, 8)
        pltpu.async_copy(idx_hbm.at[pl.ds(rb, CA * K // 128), :], ibufs[p],
                         isem)

    def wait_idx(p):
        pltpu.make_async_copy(idx_hbm.at[pl.ds(0, CA * K // 128), :],
                              ibufs[p], isem).wait()

    def issue_wave(ip, pair, p):
        pltpu.async_copy(y_hbm.at[ibufs[ip].at[2 * pair]],
                         rbufs[p].at[pl.ds(0, 128), :], gsems[p])
        pltpu.async_copy(y_hbm.at[ibufs[ip].at[2 * pair + 1]],
                         rbufs[p].at[pl.ds(128, 128), :], gsems[p])

    def wait_wave(p):
        pltpu.make_async_copy(y_hbm.at[pl.ds(0, WA), :], rbufs[p],
                              gsems[p]).wait()

    def issue_sp(g, p):
        ab = pl.multiple_of(base + g * CA, CA)
        pltpu.async_copy(sp_hbm.at[pl.ds(ab, CA), :], spbufs[p], sps)

    def wait_sp(p):
        pltpu.make_async_copy(sp_hbm.at[pl.ds(0, CA), :], spbufs[p], sps).wait()

    def issue_out(g, p):
        ab = pl.multiple_of(base + g * CA, CA)
        pltpu.async_copy(acbufs[p], s_hbm.at[pl.ds(ab, CA), :], osems[p])

    def wait_out(p):
        pltpu.make_async_copy(acbufs[p], s_hbm.at[pl.ds(0, CA), :],
                              osems[p]).wait()

    def compute_wave(w, wp, gp):
        rb, spb, acb = rbufs[wp], spbufs[gp], acbufs[gp]

        def atom_body(ai, _):
            a = w * 16 + ai
            sp = [spb[a, pl.ds(c * 16, 16)] for c in range(4)]
            acc = [jnp.zeros((16,), jnp.float32) for _ in range(4)]
            rbase = ai * K
            for k in range(K):
                for c in range(4):
                    row = rb[rbase + k, pl.ds(c * 16, 16)]
                    acc[c] = acc[c] + jnp.maximum(sp[c] + row, 0.0)
            for c in range(4):
                acb[a, pl.ds(c * 16, 16)] = acc[c]
            return 0

        lax.fori_loop(0, 16, atom_body, 0)

    def do_group(g, gp, last):
        wait_sp(gp)
        if not last:
            issue_idx(g + 1, 1 - gp)
            issue_sp(g + 1, 1 - gp)
        for w in range(4):
            wp = w % 2
            if w < 3:
                issue_wave(gp, w + 1, 1 - wp)
            elif not last:
                wait_idx(1 - gp)
                issue_wave(1 - gp, 0, 1 - wp)
            wait_wave(wp)
            compute_wave(w, wp, gp)
        issue_out(g, gp)

    issue_idx(0, 0)
    wait_idx(0)
    issue_sp(0, 0)
    issue_wave(0, 0, 0)

    def pair_body(t, _):
        @pl.when(t > 0)
        def _():
            wait_out(0)

        do_group(2 * t, 0, False)

        @pl.when(t > 0)
        def _():
            wait_out(1)

        do_group(2 * t + 1, 1, False)
        return 0

    lax.fori_loop(0, (groups - 1) // 2, pair_body, 0)

    @pl.when(groups > 1)
    def _():
        wait_out(0)

    do_group(groups - 1, 0, True)

    @pl.when(groups > 1)
    def _():
        wait_out(1)

    wait_out(0)


G_C0 = 39
G_C1 = 11


def _sc_gather_relu_sum(n_pad, y, sp, idx2d):
    total_groups = n_pad // CA
    assert NS * (G_C0 + G_C1) == total_groups
    assert G_C0 % 2 == 1 and G_C1 % 2 == 1
    mesh = plsc.VectorSubcoreMesh(core_axis_name="c", subcore_axis_name="s")
    return pl.kernel(
        functools.partial(_sc_body, G_C0, G_C1),
        out_type=jax.ShapeDtypeStruct((n_pad, F), jnp.float32),
        mesh=mesh,
        compiler_params=pltpu.CompilerParams(use_tc_tiling_on_sc=False),
        scratch_types=[
            pltpu.VMEM((CA * K // 128, 128), jnp.int32),
            pltpu.VMEM((CA * K // 128, 128), jnp.int32),
            pltpu.VMEM((256, F), jnp.float32),
            pltpu.VMEM((256, F), jnp.float32),
            pltpu.VMEM((CA, F), jnp.float32),
            pltpu.VMEM((CA, F), jnp.float32),
            pltpu.VMEM((CA, F), jnp.float32),
            pltpu.VMEM((CA, F), jnp.float32),
            pltpu.SemaphoreType.DMA,
            pltpu.SemaphoreType.DMA,
            pltpu.SemaphoreType.DMA,
            pltpu.SemaphoreType.DMA,
            pltpu.SemaphoreType.DMA,
            pltpu.SemaphoreType.DMA,
        ],
    )(y, sp, idx2d)


def kernel(atom_fea, nbr_fea, nbr_fea_idx, params):
    n = atom_fea.shape[0]
    n_pad = -(-n // (NW * CA)) * (NW * CA)
    assert n_pad % BN == 0

    af = jnp.pad(atom_fea, ((0, n_pad - n), (0, 128 - atom_fea.shape[1])))
    Wa = jnp.pad(params['W_atom'], ((0, 128 - params['W_atom'].shape[0]), (0, 0)))
    ba = params['b_atom'][None, :]
    idx2d = jnp.pad(nbr_fea_idx, ((0, n_pad - n), (0, 0))).reshape(-1, 128)

    def pre_w(l):
        c = params['conv%d' % l]
        return c['Wm1'][:F], c['bm1'][None, :], c['Wm1'][F:]

    def post_w(l):
        c = params['conv%d' % l]
        return (c['Wm2'], c['bm2'][None, :], c['Wu1'][:F], c['Wu1'][F:],
                c['bu1'][None, :], c['Wu2'], c['bu2'][None, :])

    x, sp, y = _tc_embed_pre(n_pad, af, Wa, ba, *pre_w(0))
    for l in range(2):
        s = _sc_gather_relu_sum(n_pad, y, sp, idx2d)
        x, sp, y = _tc_mid(n_pad, x, s, *post_w(l), *pre_w(l + 1))
    s = _sc_gather_relu_sum(n_pad, y, sp, idx2d)
    props, feat, xn = _tc_final(
        n_pad, x, s, *post_w(2),
        params['We1'], params['be1'][None, :],
        params['We2'], params['be2'][None, :],
        params['Wp'], params['bp'][None, :])

    return jnp.concatenate([props[:n], feat[:n], xn[:n]], axis=-1)


# speedup vs baseline: 2.3484x; 1.1855x over previous
import functools

import jax
import jax.numpy as jnp
import numpy as np
from jax import lax
from jax.experimental import pallas as pl
from jax.experimental.pallas import tpu as pltpu
from jax.experimental.pallas import tpu_sc as plsc

F = 64
K = 16
BN = 2048
NC = 2
NS = 16
NW = NC * NS
CA = 64
IDX_ROWS = (CA * K) // 128


def _full_spec(shape):
    return pl.BlockSpec(shape, lambda i: (0,) * len(shape))


def _row_spec(cols):
    return pl.BlockSpec((BN, cols), lambda i: (i, 0))


def _embed_pre_body(af, Wa, ba, W1a, b1, W1b, x_o, sp_o, y_o):
    x = jnp.dot(af[...], Wa[...], preferred_element_type=jnp.float32) + ba[...]
    x_o[...] = x
    sp_o[...] = jnp.dot(x, W1a[...], preferred_element_type=jnp.float32) + b1[...]
    y_o[...] = jnp.dot(x, W1b[...], preferred_element_type=jnp.float32)


def _update(x, s, Wm2, bm2, Wu1a, Wu1b, bu1, Wu2, bu2):
    agg = jnp.dot(s, Wm2, preferred_element_type=jnp.float32) + 16.0 * bm2
    h = jnp.maximum(
        jnp.dot(x, Wu1a, preferred_element_type=jnp.float32)
        + jnp.dot(agg, Wu1b, preferred_element_type=jnp.float32) + bu1, 0.0)
    upd = jnp.dot(h, Wu2, preferred_element_type=jnp.float32) + bu2
    return jnp.maximum(x + upd, 0.0)


def _mid_body(x, s, Wm2, bm2, Wu1a, Wu1b, bu1, Wu2, bu2, W1a, b1, W1b,
              xn_o, sp_o, y_o):
    xn = _update(x[...], s[...], Wm2[...], bm2[...], Wu1a[...], Wu1b[...],
                 bu1[...], Wu2[...], bu2[...])
    xn_o[...] = xn
    sp_o[...] = jnp.dot(xn, W1a[...], preferred_element_type=jnp.float32) + b1[...]
    y_o[...] = jnp.dot(xn, W1b[...], preferred_element_type=jnp.float32)


def _final_body(x, s, Wm2, bm2, Wu1a, Wu1b, bu1, Wu2, bu2,
                We1, be1, We2, be2, Wp, bp, out_o):
    xn = _update(x[...], s[...], Wm2[...], bm2[...], Wu1a[...], Wu1b[...],
                 bu1[...], Wu2[...], bu2[...])
    f1 = jnp.maximum(
        jnp.dot(xn, We1[...], preferred_element_type=jnp.float32) + be1[...], 0.0)
    feat = jnp.dot(f1, We2[...], preferred_element_type=jnp.float32) + be2[...]
    props = jnp.dot(feat, Wp[...], preferred_element_type=jnp.float32) + bp[...]
    out_o[...] = jnp.concatenate([props, feat, xn], axis=-1)


def _tc_embed_pre(n_pad, af, Wa, ba, W1a, b1, W1b):
    nb = n_pad // BN
    out = jax.ShapeDtypeStruct((n_pad, F), jnp.float32)
    out_y = jax.ShapeDtypeStruct((n_pad, F), jnp.float32)
    return pl.pallas_call(
        _embed_pre_body,
        grid=(nb,),
        in_specs=[_row_spec(128), _full_spec((128, F)), _full_spec((1, F)),
                  _full_spec((F, F)), _full_spec((1, F)), _full_spec((F, F))],
        out_specs=[_row_spec(F), _row_spec(F), _row_spec(F)],
        out_shape=[out, out, out_y],
    )(af, Wa, ba, W1a, b1, W1b)


def _tc_mid(n_pad, x, s, Wm2, bm2, Wu1a, Wu1b, bu1, Wu2, bu2, W1a, b1, W1b):
    nb = n_pad // BN
    out = jax.ShapeDtypeStruct((n_pad, F), jnp.float32)
    out_y = jax.ShapeDtypeStruct((n_pad, F), jnp.float32)
    wspec = _full_spec((F, F))
    bspec = _full_spec((1, F))
    return pl.pallas_call(
        _mid_body,
        grid=(nb,),
        in_specs=[_row_spec(F), _row_spec(F),
                  wspec, bspec, wspec, wspec, bspec, wspec, bspec,
                  wspec, bspec, wspec],
        out_specs=[_row_spec(F), _row_spec(F), _row_spec(F)],
        out_shape=[out, out, out_y],
    )(x, s, Wm2, bm2, Wu1a, Wu1b, bu1, Wu2, bu2, W1a, b1, W1b)


def _tc_final(n, x, s, Wm2, bm2, Wu1a, Wu1b, bu1, Wu2, bu2,
              We1, be1, We2, be2, Wp, bp):
    bnf = 2000
    assert n % bnf == 0
    wspec = _full_spec((F, F))
    bspec = _full_spec((1, F))
    rspec = lambda cols: pl.BlockSpec((bnf, cols), lambda i: (i, 0))
    return pl.pallas_call(
        _final_body,
        grid=(n // bnf,),
        in_specs=[rspec(F), rspec(F),
                  wspec, bspec, wspec, wspec, bspec, wspec, bspec,
                  _full_spec((F, 128)), _full_spec((1, 128)),
                  _full_spec((128, F)), bspec,
                  _full_spec((F, 4)), _full_spec((1, 4))],
        out_specs=[rspec(132)],
        out_shape=[jax.ShapeDtypeStruct((n, 132), jnp.float32)],
    )(x, s, Wm2, bm2, Wu1a, Wu1b, bu1, Wu2, bu2, We1, be1, We2, be2, Wp, bp)[0]


def _sc_body(G0, G1, y_hbm, sp_hbm, idx_hbm, s_hbm,
             i0, i1, r0, r1, sp0, sp1, ac0, ac1,
             isem, g0, g1, sps, o0, o1):
    c = lax.axis_index("c")
    s = lax.axis_index("s")
    is0 = c == 0
    groups = jnp.where(is0, G0, G1)
    base = jnp.where(is0, s * (G0 * CA), NS * G0 * CA + s * (G1 * CA))

    ibufs, rbufs, gsems = [i0, i1], [r0, r1], [g0, g1]
    spbufs, acbufs, osems = [sp0, sp1], [ac0, ac1], [o0, o1]
    WA = 256

    def issue_idx(g, p):
        rb = pl.multiple_of((base + g * CA) // 8, 8)
        pltpu.async_copy(idx_hbm.at[pl.ds(rb, CA * K // 128), :], ibufs[p],
                         isem)

    def wait_idx(p):
        pltpu.make_async_copy(idx_hbm.at[pl.ds(0, CA * K // 128), :],
                              ibufs[p], isem).wait()

    def issue_wave(ip, pair, p):
        pltpu.async_copy(y_hbm.at[ibufs[ip].at[2 * pair]],
                         rbufs[p].at[pl.ds(0, 128), :], gsems[p])
        pltpu.async_copy(y_hbm.at[ibufs[ip].at[2 * pair + 1]],
                         rbufs[p].at[pl.ds(128, 128), :], gsems[p])

    def wait_wave(p):
        pltpu.make_async_copy(y_hbm.at[pl.ds(0, WA), :], rbufs[p],
                              gsems[p]).wait()

    def issue_sp(g, p):
        ab = pl.multiple_of(base + g * CA, CA)
        pltpu.async_copy(sp_hbm.at[pl.ds(ab, CA), :], spbufs[p], sps)

    def wait_sp(p):
        pltpu.make_async_copy(sp_hbm.at[pl.ds(0, CA), :], spbufs[p], sps).wait()

    def issue_out(g, p):
        ab = pl.multiple_of(base + g * CA, CA)
        pltpu.async_copy(acbufs[p], s_hbm.at[pl.ds(ab, CA), :], osems[p])

    def wait_out(p):
        pltpu.make_async_copy(acbufs[p], s_hbm.at[pl.ds(0, CA), :],
                              osems[p]).wait()

    def compute_wave(w, wp, gp):
        rb, spb, acb = rbufs[wp], spbufs[gp], acbufs[gp]

        def atom_body(ai, _):
            a = w * 16 + ai
            sp = [spb[a, pl.ds(c * 16, 16)] for c in range(4)]
            acc = [jnp.zeros((16,), jnp.float32) for _ in range(4)]
            rbase = ai * K
            for k in range(K):
                for c in range(4):
                    row = rb[rbase + k, pl.ds(c * 16, 16)]
                    acc[c] = acc[c] + jnp.maximum(sp[c] + row, 0.0)
            for c in range(4):
                acb[a, pl.ds(c * 16, 16)] = acc[c]
            return 0

        lax.fori_loop(0, 16, atom_body, 0)

    def do_group(g, gp, last):
        wait_sp(gp)
        if not last:
            issue_idx(g + 1, 1 - gp)
            issue_sp(g + 1, 1 - gp)
        for w in range(4):
            wp = w % 2
            if w < 3:
                issue_wave(gp, w + 1, 1 - wp)
            elif not last:
                wait_idx(1 - gp)
                issue_wave(1 - gp, 0, 1 - wp)
            wait_wave(wp)
            compute_wave(w, wp, gp)
        issue_out(g, gp)

    issue_idx(0, 0)
    wait_idx(0)
    issue_sp(0, 0)
    issue_wave(0, 0, 0)

    def pair_body(t, _):
        @pl.when(t > 0)
        def _():
            wait_out(0)

        do_group(2 * t, 0, False)

        @pl.when(t > 0)
        def _():
            wait_out(1)

        do_group(2 * t + 1, 1, False)
        return 0

    lax.fori_loop(0, (groups - 1) // 2, pair_body, 0)

    @pl.when(groups > 1)
    def _():
        wait_out(0)

    do_group(groups - 1, 0, True)

    @pl.when(groups > 1)
    def _():
        wait_out(1)

    wait_out(0)


G_C0 = 45
G_C1 = 5


def _sc_gather_relu_sum(n_pad, y, sp, idx2d):
    total_groups = n_pad // CA
    assert NS * (G_C0 + G_C1) == total_groups
    assert G_C0 % 2 == 1 and G_C1 % 2 == 1
    mesh = plsc.VectorSubcoreMesh(core_axis_name="c", subcore_axis_name="s")
    return pl.kernel(
        functools.partial(_sc_body, G_C0, G_C1),
        out_type=jax.ShapeDtypeStruct((n_pad, F), jnp.float32),
        mesh=mesh,
        compiler_params=pltpu.CompilerParams(use_tc_tiling_on_sc=False),
        scratch_types=[
            pltpu.VMEM((CA * K // 128, 128), jnp.int32),
            pltpu.VMEM((CA * K // 128, 128), jnp.int32),
            pltpu.VMEM((256, F), jnp.float32),
            pltpu.VMEM((256, F), jnp.float32),
            pltpu.VMEM((CA, F), jnp.float32),
            pltpu.VMEM((CA, F), jnp.float32),
            pltpu.VMEM((CA, F), jnp.float32),
            pltpu.VMEM((CA, F), jnp.float32),
            pltpu.SemaphoreType.DMA,
            pltpu.SemaphoreType.DMA,
            pltpu.SemaphoreType.DMA,
            pltpu.SemaphoreType.DMA,
            pltpu.SemaphoreType.DMA,
            pltpu.SemaphoreType.DMA,
        ],
    )(y, sp, idx2d)


def kernel(atom_fea, nbr_fea, nbr_fea_idx, params):
    n = atom_fea.shape[0]
    n_pad = -(-n // (NW * CA)) * (NW * CA)
    assert n_pad % BN == 0

    af = jnp.pad(atom_fea, ((0, n_pad - n), (0, 128 - atom_fea.shape[1])))
    Wa = jnp.pad(params['W_atom'], ((0, 128 - params['W_atom'].shape[0]), (0, 0)))
    ba = params['b_atom'][None, :]
    idx2d = jnp.pad(nbr_fea_idx, ((0, n_pad - n), (0, 0))).reshape(-1, 128)

    def pre_w(l):
        c = params['conv%d' % l]
        return c['Wm1'][:F], c['bm1'][None, :], c['Wm1'][F:]

    def post_w(l):
        c = params['conv%d' % l]
        return (c['Wm2'], c['bm2'][None, :], c['Wu1'][:F], c['Wu1'][F:],
                c['bu1'][None, :], c['Wu2'], c['bu2'][None, :])

    x, sp, y = _tc_embed_pre(n_pad, af, Wa, ba, *pre_w(0))
    for l in range(2):
        s = _sc_gather_relu_sum(n_pad, y, sp, idx2d)
        x, sp, y = _tc_mid(n_pad, x, s, *post_w(l), *pre_w(l + 1))
    s = _sc_gather_relu_sum(n_pad, y, sp, idx2d)
    return _tc_final(
        n, x, s, *post_w(2),
        params['We1'], params['be1'][None, :],
        params['We2'], params['be2'][None, :],
        params['Wp'], params['bp'][None, :])
